# Initial kernel scaffold; baseline (speedup 1.0000x reference)
import jax
import jax.numpy as jnp
from jax.experimental import pallas as pl


def kernel(x, edge_index, edge_weight, W1, W2, W3, W4, W5, W6):
    raise NotImplementedError("write your pallas kernel here")


# speedup vs baseline: 3.1363x; 3.1363x over previous
import functools

import jax
import jax.numpy as jnp
from jax import lax
from jax.experimental import pallas as pl
from jax.experimental.pallas import tpu as pltpu
from jax.experimental.pallas import tpu_sc as plsc

N = 10000
E = 320000
NC, NS, LANES = 2, 16, 16
NW = NC * NS
EPW = E // NW
KCH = 80
NCHUNK = EPW // KCH
ROWS_PT = 624
ROWS_LAST = N - 15 * ROWS_PT


def _spmm_sc(support, colx, rowx, ew, dd):
    grp = dd // LANES
    mesh = plsc.VectorSubcoreMesh(core_axis_name="c", subcore_axis_name="s")
    zeros = jnp.zeros((N, dd), jnp.float32)

    @functools.partial(
        pl.kernel,
        out_type=jax.ShapeDtypeStruct((NC, N, dd), jnp.float32),
        mesh=mesh,
        scratch_types=[
            pltpu.VMEM((KCH,), jnp.int32),
            pltpu.VMEM((KCH,), jnp.int32),
            pltpu.VMEM((KCH,), jnp.float32),
            pltpu.VMEM((KCH, dd), jnp.float32),
            pltpu.VMEM_SHARED((N, dd), jnp.float32),
            pltpu.SemaphoreType.DMA,
        ],
    )
    def k(sup_hbm, col_hbm, row_hbm, w_hbm, z_hbm, out_hbm,
          col_v, row_v, w_v, msg_v, acc, sem):
        c = lax.axis_index("c")
        s = lax.axis_index("s")
        wid = c * NS + s
        @pl.when(s < NS - 1)
        def _():
            pltpu.sync_copy(z_hbm.at[pl.ds(s * ROWS_PT, ROWS_PT)],
                            acc.at[pl.ds(s * ROWS_PT, ROWS_PT)])

        @pl.when(s == NS - 1)
        def _():
            pltpu.sync_copy(z_hbm.at[pl.ds(15 * ROWS_PT, ROWS_LAST)],
                            acc.at[pl.ds(15 * ROWS_PT, ROWS_LAST)])

        plsc.subcore_barrier()

        def chunk(i, carry):
            base = wid * EPW + i * KCH
            pltpu.sync_copy(col_hbm.at[pl.ds(base, KCH)], col_v)
            pltpu.sync_copy(row_hbm.at[pl.ds(base, KCH)], row_v)
            pltpu.sync_copy(w_hbm.at[pl.ds(base, KCH)], w_v)
            pltpu.async_copy(sup_hbm.at[col_v], msg_v, sem).wait()

            def edge_group(g, cc):
                wv = w_v[pl.ds(g * LANES, LANES)]
                for t in range(LANES):
                    wk = wv[t]
                    kk = g * LANES + t
                    for j in range(grp):
                        sl = pl.ds(j * LANES, LANES)
                        msg_v[kk, sl] = msg_v[kk, sl] * wk
                return cc

            lax.fori_loop(0, KCH // LANES, edge_group, 0)
            pltpu.sync_copy(msg_v, acc.at[row_v], add=True)
            return carry

        lax.fori_loop(0, NCHUNK, chunk, 0)
        plsc.subcore_barrier()

        @pl.when(s < NS - 1)
        def _():
            pltpu.sync_copy(acc.at[pl.ds(s * ROWS_PT, ROWS_PT)],
                            out_hbm.at[c, pl.ds(s * ROWS_PT, ROWS_PT)])

        @pl.when(s == NS - 1)
        def _():
            pltpu.sync_copy(acc.at[pl.ds(15 * ROWS_PT, ROWS_LAST)],
                            out_hbm.at[c, pl.ds(15 * ROWS_PT, ROWS_LAST)])

    return k(support, colx, rowx, ew, zeros)


MM_RB = 1000


def _tc_mm(parts, ws, act, nsplit):
    nin = len(parts)
    s_dim = parts[0].shape[0]
    dout = ws[0].shape[1]
    dh = dout // nsplit
    in_specs = []
    for p in parts:
        din = p.shape[2]
        in_specs.append(
            pl.BlockSpec((s_dim, MM_RB, din), lambda i: (0, i, 0)))
    for w in ws:
        in_specs.append(pl.BlockSpec(w.shape, lambda i: (0, 0)))

    def body(*refs):
        ins, wrefs, outs = refs[:nin], refs[nin:2 * nin], refs[2 * nin:]
        acc = None
        for t in range(nin):
            a = ins[t][0] if s_dim == 1 else ins[t][0] + ins[t][1]
            prod = jnp.dot(a, wrefs[t][...], preferred_element_type=jnp.float32)
            acc = prod if acc is None else acc + prod
        acc = act(acc)
        for u in range(nsplit):
            outs[u][...] = acc[:, u * dh:(u + 1) * dh]

    return pl.pallas_call(
        body,
        grid=(N // MM_RB,),
        in_specs=in_specs,
        out_specs=[pl.BlockSpec((MM_RB, dh), lambda i: (i, 0))] * nsplit,
        out_shape=[jax.ShapeDtypeStruct((N, dh), jnp.float32)] * nsplit,
    )(*parts, *ws)


def _add2(p):
    d = p.shape[2]

    def body(pr, o):
        o[...] = pr[0] + pr[1]

    return pl.pallas_call(
        body,
        grid=(N // MM_RB,),
        in_specs=[pl.BlockSpec((2, MM_RB, d), lambda i: (0, i, 0))],
        out_specs=pl.BlockSpec((MM_RB, d), lambda i: (i, 0)),
        out_shape=jax.ShapeDtypeStruct((N, d), jnp.float32),
    )(p)


ADJ_BM, ADJ_BN = 400, 10000


def _adj(zi, zh):
    dims = (((1,), (1,)), ((), ()))

    def body(zi_i, zh_i, zi_j, zh_j, o):
        s1 = lax.dot_general(zi_i[...], zi_j[...], dims,
                             preferred_element_type=jnp.float32)
        s2 = lax.dot_general(zh_i[...], zh_j[...], dims,
                             preferred_element_type=jnp.float32)
        o[...] = jax.nn.sigmoid(s1) + jax.nn.sigmoid(s2)

    return pl.pallas_call(
        body,
        grid=(N // ADJ_BM, N // ADJ_BN),
        in_specs=[
            pl.BlockSpec((ADJ_BM, zi.shape[1]), lambda i, j: (i, 0)),
            pl.BlockSpec((ADJ_BM, zh.shape[1]), lambda i, j: (i, 0)),
            pl.BlockSpec((ADJ_BN, zi.shape[1]), lambda i, j: (j, 0)),
            pl.BlockSpec((ADJ_BN, zh.shape[1]), lambda i, j: (j, 0)),
        ],
        out_specs=pl.BlockSpec((ADJ_BM, ADJ_BN), lambda i, j: (i, j)),
        out_shape=jax.ShapeDtypeStruct((N, N), jnp.float32),
    )(zi, zh, zi, zh)


def kernel(x, edge_index, edge_weight, W1, W2, W3, W4, W5, W6):
    rowx = edge_index[0]
    colx = edge_index[1]
    ew = edge_weight
    tanh = jnp.tanh
    ident = lambda a: a
    w3p = jnp.pad(W3, ((0, 0), (0, 108)))
    w4p = jnp.pad(W4, ((0, 108), (0, 0)))

    t1, = _tc_mm([x[None]], [W1], tanh, 1)
    z1p = _spmm_sc(t1, colx, rowx, ew, 128)
    t2l, t2r = _tc_mm([z1p], [W2], tanh, 2)
    z2lp = _spmm_sc(t2l, colx, rowx, ew, 128)
    z2rp = _spmm_sc(t2r, colx, rowx, ew, 128)
    t3, = _tc_mm([z2lp, z2rp], [w3p[:128], w3p[128:]], ident, 1)
    zip_ = _spmm_sc(t3, colx, rowx, ew, 128)
    t4l, t4r = _tc_mm([zip_], [w4p], tanh, 2)
    h1lp = _spmm_sc(t4l, colx, rowx, ew, 128)
    h1rp = _spmm_sc(t4r, colx, rowx, ew, 128)
    t5, = _tc_mm([h1lp, h1rp], [W5[:128], W5[128:]], tanh, 1)
    h2p = _spmm_sc(t5, colx, rowx, ew, 128)
    t6, = _tc_mm([h2p], [W6], tanh, 1)
    zhp = _spmm_sc(t6, colx, rowx, ew, 128)

    zi_pad = _add2(zip_)
    z_hat = _add2(zhp)
    z_igae = zi_pad[:, :20]
    adj_hat = _adj(zi_pad, z_hat)
    return (z_igae, z_hat, adj_hat)


# speedup vs baseline: 7.5495x; 2.4071x over previous
import functools

import jax
import jax.numpy as jnp
from jax import lax
from jax.experimental import pallas as pl
from jax.experimental.pallas import tpu as pltpu
from jax.experimental.pallas import tpu_sc as plsc

N = 10000
E = 320000
NC, NS, LANES = 2, 16, 16
NW = NC * NS
EPW = E // NW
KCH = 80
NCHUNK = EPW // KCH
ROWS_PT = 624
ROWS_LAST = N - 15 * ROWS_PT


MRING = 4
IRING = 5


def _spmm_sc(support, colx, rowx, ew, dd):
    grp = dd // LANES
    mesh = plsc.VectorSubcoreMesh(core_axis_name="c", subcore_axis_name="s")
    zeros = jnp.zeros((N, dd), jnp.float32)

    scratch = (
        [pltpu.VMEM((KCH,), jnp.int32)] * IRING +
        [pltpu.VMEM((KCH,), jnp.int32)] * IRING +
        [pltpu.VMEM((KCH,), jnp.float32)] * IRING +
        [pltpu.VMEM((KCH, dd), jnp.float32)] * MRING +
        [pltpu.VMEM_SHARED((N, dd), jnp.float32)] +
        [pltpu.SemaphoreType.DMA] * (IRING + 2 * MRING)
    )

    @functools.partial(
        pl.kernel,
        out_type=jax.ShapeDtypeStruct((NC, N, dd), jnp.float32),
        mesh=mesh,
        scratch_types=scratch,
    )
    def k(sup_hbm, col_hbm, row_hbm, w_hbm, z_hbm, out_hbm, *scr):
        col = scr[0:IRING]
        row = scr[IRING:2 * IRING]
        wgt = scr[2 * IRING:3 * IRING]
        base = 3 * IRING
        msg = scr[base:base + MRING]
        acc = scr[base + MRING]
        semi = scr[base + MRING + 1:base + MRING + 1 + IRING]
        semg = scr[base + MRING + 1 + IRING:base + MRING + 1 + IRING + MRING]
        sems = scr[base + MRING + 1 + IRING + MRING:]
        c = lax.axis_index("c")
        s = lax.axis_index("s")
        wid = c * NS + s
        ebase = wid * EPW

        def issue_idx(i, sl):
            bb = ebase + i * KCH
            pltpu.async_copy(col_hbm.at[pl.ds(bb, KCH)], col[sl], semi[sl])
            pltpu.async_copy(row_hbm.at[pl.ds(bb, KCH)], row[sl], semi[sl])
            pltpu.async_copy(w_hbm.at[pl.ds(bb, KCH)], wgt[sl], semi[sl])

        def wait_idx(sl):
            for src, dst in ((col_hbm, col[sl]), (row_hbm, row[sl]),
                             (w_hbm, wgt[sl])):
                pltpu.make_async_copy(src.at[pl.ds(0, KCH)], dst,
                                      semi[sl]).wait()

        def turn(i, ms, isl, static_i):
            ms1 = (ms + 1) % MRING
            msd = (ms + 2) % MRING
            is1 = (isl + 1) % IRING
            is2 = (isl + 2) % IRING
            isd = (isl + 3) % IRING

            def drain():
                pltpu.make_async_copy(msg[msd], acc.at[row[isd]],
                                      sems[msd]).wait()

            if static_i:
                if i >= 2:
                    drain()
            else:
                pl.when(i >= 2)(drain)

            def next_gather():
                wait_idx(is1)
                pltpu.async_copy(sup_hbm.at[col[is1]], msg[ms1], semg[ms1])

            def prefetch_idx():
                issue_idx(i + 2, is2)

            if static_i:
                if i + 1 < NCHUNK:
                    next_gather()
                if i + 2 < NCHUNK:
                    prefetch_idx()
            else:
                next_gather()
                prefetch_idx()

            pltpu.make_async_copy(sup_hbm.at[col[isl]], msg[ms],
                                  semg[ms]).wait()

            def edge_group(g, cc):
                wv = wgt[isl][pl.ds(g * LANES, LANES)]
                for t in range(LANES):
                    wk = wv[t]
                    kk = g * LANES + t
                    for j in range(grp):
                        sl = pl.ds(j * LANES, LANES)
                        msg[ms][kk, sl] = msg[ms][kk, sl] * wk
                return cc

            lax.fori_loop(0, KCH // LANES, edge_group, 0)
            pltpu.async_copy(msg[ms], acc.at[row[isl]], sems[ms], add=True)

        @pl.when(s < NS - 1)
        def _():
            pltpu.sync_copy(z_hbm.at[pl.ds(s * ROWS_PT, ROWS_PT)],
                            acc.at[pl.ds(s * ROWS_PT, ROWS_PT)])

        @pl.when(s == NS - 1)
        def _():
            pltpu.sync_copy(z_hbm.at[pl.ds(15 * ROWS_PT, ROWS_LAST)],
                            acc.at[pl.ds(15 * ROWS_PT, ROWS_LAST)])

        plsc.subcore_barrier()

        issue_idx(0, 0)
        issue_idx(1, 1)
        wait_idx(0)
        pltpu.async_copy(sup_hbm.at[col[0]], msg[0], semg[0])

        UNROLL = 20

        def outer(m, carry):
            for b in range(UNROLL):
                turn(m * UNROLL + b, b % MRING, b % IRING, False)
            return carry

        lax.fori_loop(0, (NCHUNK - IRING) // UNROLL, outer, 0)
        for t in range(NCHUNK - IRING, NCHUNK):
            turn(t, t % MRING, t % IRING, True)
        for t in (NCHUNK - 2, NCHUNK - 1):
            pltpu.make_async_copy(msg[t % MRING], acc.at[row[t % IRING]],
                                  sems[t % MRING]).wait()
        plsc.subcore_barrier()

        @pl.when(s < NS - 1)
        def _():
            pltpu.sync_copy(acc.at[pl.ds(s * ROWS_PT, ROWS_PT)],
                            out_hbm.at[c, pl.ds(s * ROWS_PT, ROWS_PT)])

        @pl.when(s == NS - 1)
        def _():
            pltpu.sync_copy(acc.at[pl.ds(15 * ROWS_PT, ROWS_LAST)],
                            out_hbm.at[c, pl.ds(15 * ROWS_PT, ROWS_LAST)])

    return k(support, colx, rowx, ew, zeros)


MM_RB = 1000


def _tc_mm(parts, ws, act, nsplit):
    nin = len(parts)
    s_dim = parts[0].shape[0]
    dout = ws[0].shape[1]
    dh = dout // nsplit
    in_specs = []
    for p in parts:
        din = p.shape[2]
        in_specs.append(
            pl.BlockSpec((s_dim, MM_RB, din), lambda i: (0, i, 0)))
    for w in ws:
        in_specs.append(pl.BlockSpec(w.shape, lambda i: (0, 0)))

    def body(*refs):
        ins, wrefs, outs = refs[:nin], refs[nin:2 * nin], refs[2 * nin:]
        acc = None
        for t in range(nin):
            a = ins[t][0] if s_dim == 1 else ins[t][0] + ins[t][1]
            prod = jnp.dot(a, wrefs[t][...], preferred_element_type=jnp.float32)
            acc = prod if acc is None else acc + prod
        acc = act(acc)
        for u in range(nsplit):
            outs[u][...] = acc[:, u * dh:(u + 1) * dh]

    return pl.pallas_call(
        body,
        grid=(N // MM_RB,),
        in_specs=in_specs,
        out_specs=[pl.BlockSpec((MM_RB, dh), lambda i: (i, 0))] * nsplit,
        out_shape=[jax.ShapeDtypeStruct((N, dh), jnp.float32)] * nsplit,
    )(*parts, *ws)


def _add2(p):
    d = p.shape[2]

    def body(pr, o):
        o[...] = pr[0] + pr[1]

    return pl.pallas_call(
        body,
        grid=(N // MM_RB,),
        in_specs=[pl.BlockSpec((2, MM_RB, d), lambda i: (0, i, 0))],
        out_specs=pl.BlockSpec((MM_RB, d), lambda i: (i, 0)),
        out_shape=jax.ShapeDtypeStruct((N, d), jnp.float32),
    )(p)


ADJ_BM, ADJ_BN = 400, 10000


def _adj(zi, zh):
    dims = (((1,), (1,)), ((), ()))

    def body(zi_i, zh_i, zi_j, zh_j, o):
        s1 = lax.dot_general(zi_i[...], zi_j[...], dims,
                             preferred_element_type=jnp.float32)
        s2 = lax.dot_general(zh_i[...], zh_j[...], dims,
                             preferred_element_type=jnp.float32)
        o[...] = jax.nn.sigmoid(s1) + jax.nn.sigmoid(s2)

    return pl.pallas_call(
        body,
        grid=(N // ADJ_BM, N // ADJ_BN),
        in_specs=[
            pl.BlockSpec((ADJ_BM, zi.shape[1]), lambda i, j: (i, 0)),
            pl.BlockSpec((ADJ_BM, zh.shape[1]), lambda i, j: (i, 0)),
            pl.BlockSpec((ADJ_BN, zi.shape[1]), lambda i, j: (j, 0)),
            pl.BlockSpec((ADJ_BN, zh.shape[1]), lambda i, j: (j, 0)),
        ],
        out_specs=pl.BlockSpec((ADJ_BM, ADJ_BN), lambda i, j: (i, j)),
        out_shape=jax.ShapeDtypeStruct((N, N), jnp.float32),
    )(zi, zh, zi, zh)


def kernel(x, edge_index, edge_weight, W1, W2, W3, W4, W5, W6):
    rowx = edge_index[0]
    colx = edge_index[1]
    ew = edge_weight
    tanh = jnp.tanh
    ident = lambda a: a
    w3p = jnp.pad(W3, ((0, 0), (0, 108)))
    w4p = jnp.pad(W4, ((0, 108), (0, 0)))

    t1, = _tc_mm([x[None]], [W1], tanh, 1)
    z1p = _spmm_sc(t1, colx, rowx, ew, 128)
    t2l, t2r = _tc_mm([z1p], [W2], tanh, 2)
    z2lp = _spmm_sc(t2l, colx, rowx, ew, 128)
    z2rp = _spmm_sc(t2r, colx, rowx, ew, 128)
    t3, = _tc_mm([z2lp, z2rp], [w3p[:128], w3p[128:]], ident, 1)
    zip_ = _spmm_sc(t3, colx, rowx, ew, 128)
    t4l, t4r = _tc_mm([zip_], [w4p], tanh, 2)
    h1lp = _spmm_sc(t4l, colx, rowx, ew, 128)
    h1rp = _spmm_sc(t4r, colx, rowx, ew, 128)
    t5, = _tc_mm([h1lp, h1rp], [W5[:128], W5[128:]], tanh, 1)
    h2p = _spmm_sc(t5, colx, rowx, ew, 128)
    t6, = _tc_mm([h2p], [W6], tanh, 1)
    zhp = _spmm_sc(t6, colx, rowx, ew, 128)

    zi_pad = _add2(zip_)
    z_hat = _add2(zhp)
    z_igae = zi_pad[:, :20]
    adj_hat = _adj(zi_pad, z_hat)
    return (z_igae, z_hat, adj_hat)


# speedup vs baseline: 8.0794x; 1.0702x over previous
import functools

import jax
import jax.numpy as jnp
from jax import lax
from jax.experimental import pallas as pl
from jax.experimental.pallas import tpu as pltpu
from jax.experimental.pallas import tpu_sc as plsc

N = 10000
E = 320000
NC, NS, LANES = 2, 16, 16
NW = NC * NS
EPW = E // NW
KCH = 80
NCHUNK = EPW // KCH
ROWS_PT = 624
ROWS_LAST = N - 15 * ROWS_PT


MRING = 4
IRING = 5


def _spmm_sc(support, colx, rowx, ew, dd):
    grp = dd // LANES
    mesh = plsc.VectorSubcoreMesh(core_axis_name="c", subcore_axis_name="s")
    zeros = jnp.zeros((N, dd), jnp.float32)

    scratch = (
        [pltpu.VMEM((KCH,), jnp.int32)] * IRING +
        [pltpu.VMEM((KCH,), jnp.int32)] * IRING +
        [pltpu.VMEM((KCH,), jnp.float32)] * IRING +
        [pltpu.VMEM((KCH, dd), jnp.float32)] * MRING +
        [pltpu.VMEM_SHARED((N, dd), jnp.float32)] +
        [pltpu.SemaphoreType.DMA] * (IRING + 2 * MRING)
    )

    @functools.partial(
        pl.kernel,
        out_type=jax.ShapeDtypeStruct((NC, N, dd), jnp.float32),
        mesh=mesh,
        scratch_types=scratch,
    )
    def k(sup_hbm, col_hbm, row_hbm, w_hbm, z_hbm, out_hbm, *scr):
        col = scr[0:IRING]
        row = scr[IRING:2 * IRING]
        wgt = scr[2 * IRING:3 * IRING]
        base = 3 * IRING
        msg = scr[base:base + MRING]
        acc = scr[base + MRING]
        semi = scr[base + MRING + 1:base + MRING + 1 + IRING]
        semg = scr[base + MRING + 1 + IRING:base + MRING + 1 + IRING + MRING]
        sems = scr[base + MRING + 1 + IRING + MRING:]
        c = lax.axis_index("c")
        s = lax.axis_index("s")
        wid = c * NS + s
        ebase = wid * EPW

        def issue_idx(i, sl):
            bb = ebase + i * KCH
            pltpu.async_copy(col_hbm.at[pl.ds(bb, KCH)], col[sl], semi[sl])
            pltpu.async_copy(row_hbm.at[pl.ds(bb, KCH)], row[sl], semi[sl])
            pltpu.async_copy(w_hbm.at[pl.ds(bb, KCH)], wgt[sl], semi[sl])

        def wait_idx(sl):
            for src, dst in ((col_hbm, col[sl]), (row_hbm, row[sl]),
                             (w_hbm, wgt[sl])):
                pltpu.make_async_copy(src.at[pl.ds(0, KCH)], dst,
                                      semi[sl]).wait()

        def turn(i, ms, isl, static_i):
            ms2 = (ms + 2) % MRING
            is2 = (isl + 2) % IRING
            is3 = (isl + 3) % IRING

            def drain():
                pltpu.make_async_copy(msg[ms2], acc.at[row[is3]],
                                      sems[ms2]).wait()

            if static_i:
                if i >= 2:
                    drain()
            else:
                pl.when(i >= 2)(drain)

            def next_gather():
                wait_idx(is2)
                pltpu.async_copy(sup_hbm.at[col[is2]], msg[ms2], semg[ms2])

            def prefetch_idx():
                issue_idx(i + 3, is3)

            if static_i:
                if i + 2 < NCHUNK:
                    next_gather()
                if i + 3 < NCHUNK:
                    prefetch_idx()
            else:
                next_gather()
                prefetch_idx()

            pltpu.make_async_copy(sup_hbm.at[col[isl]], msg[ms],
                                  semg[ms]).wait()

            def edge_group(g, cc):
                wv = wgt[isl][pl.ds(g * LANES, LANES)]
                for t in range(LANES):
                    wk = wv[t]
                    kk = g * LANES + t
                    for j in range(grp):
                        sl = pl.ds(j * LANES, LANES)
                        msg[ms][kk, sl] = msg[ms][kk, sl] * wk
                return cc

            lax.fori_loop(0, KCH // LANES, edge_group, 0)
            pltpu.async_copy(msg[ms], acc.at[row[isl]], sems[ms], add=True)

        @pl.when(s < NS - 1)
        def _():
            pltpu.sync_copy(z_hbm.at[pl.ds(s * ROWS_PT, ROWS_PT)],
                            acc.at[pl.ds(s * ROWS_PT, ROWS_PT)])

        @pl.when(s == NS - 1)
        def _():
            pltpu.sync_copy(z_hbm.at[pl.ds(15 * ROWS_PT, ROWS_LAST)],
                            acc.at[pl.ds(15 * ROWS_PT, ROWS_LAST)])

        plsc.subcore_barrier()

        issue_idx(0, 0)
        issue_idx(1, 1)
        issue_idx(2, 2)
        wait_idx(0)
        pltpu.async_copy(sup_hbm.at[col[0]], msg[0], semg[0])
        wait_idx(1)
        pltpu.async_copy(sup_hbm.at[col[1]], msg[1], semg[1])

        UNROLL = 20

        def outer(m, carry):
            for b in range(UNROLL):
                turn(m * UNROLL + b, b % MRING, b % IRING, False)
            return carry

        lax.fori_loop(0, (NCHUNK - IRING) // UNROLL, outer, 0)
        for t in range(NCHUNK - IRING, NCHUNK):
            turn(t, t % MRING, t % IRING, True)
        for t in (NCHUNK - 2, NCHUNK - 1):
            pltpu.make_async_copy(msg[t % MRING], acc.at[row[t % IRING]],
                                  sems[t % MRING]).wait()
        plsc.subcore_barrier()

        @pl.when(s < NS - 1)
        def _():
            pltpu.sync_copy(acc.at[pl.ds(s * ROWS_PT, ROWS_PT)],
                            out_hbm.at[c, pl.ds(s * ROWS_PT, ROWS_PT)])

        @pl.when(s == NS - 1)
        def _():
            pltpu.sync_copy(acc.at[pl.ds(15 * ROWS_PT, ROWS_LAST)],
                            out_hbm.at[c, pl.ds(15 * ROWS_PT, ROWS_LAST)])

    return k(support, colx, rowx, ew, zeros)


MM_RB = 1000


def _tc_mm(parts, ws, act, nsplit):
    nin = len(parts)
    s_dim = parts[0].shape[0]
    dout = ws[0].shape[1]
    dh = dout // nsplit
    in_specs = []
    for p in parts:
        din = p.shape[2]
        in_specs.append(
            pl.BlockSpec((s_dim, MM_RB, din), lambda i: (0, i, 0)))
    for w in ws:
        in_specs.append(pl.BlockSpec(w.shape, lambda i: (0, 0)))

    def body(*refs):
        ins, wrefs, outs = refs[:nin], refs[nin:2 * nin], refs[2 * nin:]
        acc = None
        for t in range(nin):
            a = ins[t][0] if s_dim == 1 else ins[t][0] + ins[t][1]
            prod = jnp.dot(a, wrefs[t][...], preferred_element_type=jnp.float32)
            acc = prod if acc is None else acc + prod
        acc = act(acc)
        for u in range(nsplit):
            outs[u][...] = acc[:, u * dh:(u + 1) * dh]

    return pl.pallas_call(
        body,
        grid=(N // MM_RB,),
        in_specs=in_specs,
        out_specs=[pl.BlockSpec((MM_RB, dh), lambda i: (i, 0))] * nsplit,
        out_shape=[jax.ShapeDtypeStruct((N, dh), jnp.float32)] * nsplit,
    )(*parts, *ws)


def _add2(p):
    d = p.shape[2]

    def body(pr, o):
        o[...] = pr[0] + pr[1]

    return pl.pallas_call(
        body,
        grid=(N // MM_RB,),
        in_specs=[pl.BlockSpec((2, MM_RB, d), lambda i: (0, i, 0))],
        out_specs=pl.BlockSpec((MM_RB, d), lambda i: (i, 0)),
        out_shape=jax.ShapeDtypeStruct((N, d), jnp.float32),
    )(p)


ADJ_BM, ADJ_BN = 400, 10000


def _adj(zi, zh):
    dims = (((1,), (1,)), ((), ()))

    def body(zi_i, zh_i, zi_j, zh_j, o):
        s1 = lax.dot_general(zi_i[...], zi_j[...], dims,
                             preferred_element_type=jnp.float32)
        s2 = lax.dot_general(zh_i[...], zh_j[...], dims,
                             preferred_element_type=jnp.float32)
        o[...] = jax.nn.sigmoid(s1) + jax.nn.sigmoid(s2)

    return pl.pallas_call(
        body,
        grid=(N // ADJ_BM, N // ADJ_BN),
        in_specs=[
            pl.BlockSpec((ADJ_BM, zi.shape[1]), lambda i, j: (i, 0)),
            pl.BlockSpec((ADJ_BM, zh.shape[1]), lambda i, j: (i, 0)),
            pl.BlockSpec((ADJ_BN, zi.shape[1]), lambda i, j: (j, 0)),
            pl.BlockSpec((ADJ_BN, zh.shape[1]), lambda i, j: (j, 0)),
        ],
        out_specs=pl.BlockSpec((ADJ_BM, ADJ_BN), lambda i, j: (i, j)),
        out_shape=jax.ShapeDtypeStruct((N, N), jnp.float32),
    )(zi, zh, zi, zh)


def kernel(x, edge_index, edge_weight, W1, W2, W3, W4, W5, W6):
    rowx = edge_index[0]
    colx = edge_index[1]
    ew = edge_weight
    tanh = jnp.tanh
    ident = lambda a: a
    w3p = jnp.pad(W3, ((0, 0), (0, 108)))
    w4p = jnp.pad(W4, ((0, 108), (0, 0)))

    t1, = _tc_mm([x[None]], [W1], tanh, 1)
    z1p = _spmm_sc(t1, colx, rowx, ew, 128)
    t2l, t2r = _tc_mm([z1p], [W2], tanh, 2)
    z2lp = _spmm_sc(t2l, colx, rowx, ew, 128)
    z2rp = _spmm_sc(t2r, colx, rowx, ew, 128)
    t3, = _tc_mm([z2lp, z2rp], [w3p[:128], w3p[128:]], ident, 1)
    zip_ = _spmm_sc(t3, colx, rowx, ew, 128)
    t4l, t4r = _tc_mm([zip_], [w4p], tanh, 2)
    h1lp = _spmm_sc(t4l, colx, rowx, ew, 128)
    h1rp = _spmm_sc(t4r, colx, rowx, ew, 128)
    t5, = _tc_mm([h1lp, h1rp], [W5[:128], W5[128:]], tanh, 1)
    h2p = _spmm_sc(t5, colx, rowx, ew, 128)
    t6, = _tc_mm([h2p], [W6], tanh, 1)
    zhp = _spmm_sc(t6, colx, rowx, ew, 128)

    zi_pad = _add2(zip_)
    z_hat = _add2(zhp)
    z_igae = zi_pad[:, :20]
    adj_hat = _adj(zi_pad, z_hat)
    return (z_igae, z_hat, adj_hat)


# speedup vs baseline: 8.0862x; 1.0008x over previous
import functools

import jax
import jax.numpy as jnp
from jax import lax
from jax.experimental import pallas as pl
from jax.experimental.pallas import tpu as pltpu
from jax.experimental.pallas import tpu_sc as plsc

N = 10000
E = 320000
NC, NS, LANES = 2, 16, 16
NW = NC * NS
EPW = E // NW
KCH = 80
NCHUNK = EPW // KCH
ROWS_PT = 624
ROWS_LAST = N - 15 * ROWS_PT


MRING = 4
IRING = 5


def _spmm_sc(support, colx, rowx, ew, dd):
    grp = dd // LANES
    mesh = plsc.VectorSubcoreMesh(core_axis_name="c", subcore_axis_name="s")
    zeros = jnp.zeros((N, dd), jnp.float32)

    scratch = (
        [pltpu.VMEM((KCH,), jnp.int32)] * IRING +
        [pltpu.VMEM((KCH,), jnp.int32)] * IRING +
        [pltpu.VMEM((KCH,), jnp.float32)] * IRING +
        [pltpu.VMEM((KCH, dd), jnp.float32)] * MRING +
        [pltpu.VMEM_SHARED((N, dd), jnp.float32)] +
        [pltpu.SemaphoreType.DMA] * (IRING + 2 * MRING)
    )

    @functools.partial(
        pl.kernel,
        out_type=jax.ShapeDtypeStruct((NC, N, dd), jnp.float32),
        mesh=mesh,
        scratch_types=scratch,
    )
    def k(sup_hbm, col_hbm, row_hbm, w_hbm, z_hbm, out_hbm, *scr):
        col = scr[0:IRING]
        row = scr[IRING:2 * IRING]
        wgt = scr[2 * IRING:3 * IRING]
        base = 3 * IRING
        msg = scr[base:base + MRING]
        acc = scr[base + MRING]
        semi = scr[base + MRING + 1:base + MRING + 1 + IRING]
        semg = scr[base + MRING + 1 + IRING:base + MRING + 1 + IRING + MRING]
        sems = scr[base + MRING + 1 + IRING + MRING:]
        c = lax.axis_index("c")
        s = lax.axis_index("s")
        wid = c * NS + s
        ebase = wid * EPW

        def issue_idx(i, sl):
            bb = ebase + i * KCH
            pltpu.async_copy(col_hbm.at[pl.ds(bb, KCH)], col[sl], semi[sl])
            pltpu.async_copy(row_hbm.at[pl.ds(bb, KCH)], row[sl], semi[sl])
            pltpu.async_copy(w_hbm.at[pl.ds(bb, KCH)], wgt[sl], semi[sl])

        def wait_idx(sl):
            for src, dst in ((col_hbm, col[sl]), (row_hbm, row[sl]),
                             (w_hbm, wgt[sl])):
                pltpu.make_async_copy(src.at[pl.ds(0, KCH)], dst,
                                      semi[sl]).wait()

        def turn(i, ms, isl, static_i):
            ms2 = (ms + 2) % MRING
            is2 = (isl + 2) % IRING
            is3 = (isl + 3) % IRING

            def drain():
                pltpu.make_async_copy(msg[ms2], acc.at[row[is3]],
                                      sems[ms2]).wait()

            if static_i:
                if i >= 2:
                    drain()
            else:
                pl.when(i >= 2)(drain)

            def next_gather():
                wait_idx(is2)
                pltpu.async_copy(sup_hbm.at[col[is2]], msg[ms2], semg[ms2])

            def prefetch_idx():
                issue_idx(i + 3, is3)

            if static_i:
                if i + 2 < NCHUNK:
                    next_gather()
                if i + 3 < NCHUNK:
                    prefetch_idx()
            else:
                next_gather()
                prefetch_idx()

            pltpu.make_async_copy(sup_hbm.at[col[isl]], msg[ms],
                                  semg[ms]).wait()

            def edge_group(g, cc):
                wv = wgt[isl][pl.ds(g * LANES, LANES)]
                for t in range(LANES):
                    wk = wv[t]
                    kk = g * LANES + t
                    for j in range(grp):
                        sl = pl.ds(j * LANES, LANES)
                        msg[ms][kk, sl] = msg[ms][kk, sl] * wk
                return cc

            lax.fori_loop(0, KCH // LANES, edge_group, 0)
            pltpu.async_copy(msg[ms], acc.at[row[isl]], sems[ms], add=True)

        @pl.when(s < NS - 1)
        def _():
            pltpu.sync_copy(z_hbm.at[pl.ds(s * ROWS_PT, ROWS_PT)],
                            acc.at[pl.ds(s * ROWS_PT, ROWS_PT)])

        @pl.when(s == NS - 1)
        def _():
            pltpu.sync_copy(z_hbm.at[pl.ds(15 * ROWS_PT, ROWS_LAST)],
                            acc.at[pl.ds(15 * ROWS_PT, ROWS_LAST)])

        plsc.subcore_barrier()

        issue_idx(0, 0)
        issue_idx(1, 1)
        issue_idx(2, 2)
        wait_idx(0)
        pltpu.async_copy(sup_hbm.at[col[0]], msg[0], semg[0])
        wait_idx(1)
        pltpu.async_copy(sup_hbm.at[col[1]], msg[1], semg[1])

        UNROLL = 20

        def outer(m, carry):
            for b in range(UNROLL):
                turn(m * UNROLL + b, b % MRING, b % IRING, False)
            return carry

        lax.fori_loop(0, (NCHUNK - IRING) // UNROLL, outer, 0)
        for t in range(NCHUNK - IRING, NCHUNK):
            turn(t, t % MRING, t % IRING, True)
        for t in (NCHUNK - 2, NCHUNK - 1):
            pltpu.make_async_copy(msg[t % MRING], acc.at[row[t % IRING]],
                                  sems[t % MRING]).wait()
        plsc.subcore_barrier()

        @pl.when(s < NS - 1)
        def _():
            pltpu.sync_copy(acc.at[pl.ds(s * ROWS_PT, ROWS_PT)],
                            out_hbm.at[c, pl.ds(s * ROWS_PT, ROWS_PT)])

        @pl.when(s == NS - 1)
        def _():
            pltpu.sync_copy(acc.at[pl.ds(15 * ROWS_PT, ROWS_LAST)],
                            out_hbm.at[c, pl.ds(15 * ROWS_PT, ROWS_LAST)])

    return k(support, colx, rowx, ew, zeros)


MM_RB = 1000


def _tc_mm(parts, ws, act, nsplit):
    nin = len(parts)
    s_dim = parts[0].shape[0]
    dout = ws[0].shape[1]
    dh = dout // nsplit
    in_specs = []
    for p in parts:
        din = p.shape[2]
        in_specs.append(
            pl.BlockSpec((s_dim, MM_RB, din), lambda i: (0, i, 0)))
    for w in ws:
        in_specs.append(pl.BlockSpec(w.shape, lambda i: (0, 0)))

    def body(*refs):
        ins, wrefs, outs = refs[:nin], refs[nin:2 * nin], refs[2 * nin:]
        acc = None
        for t in range(nin):
            a = ins[t][0] if s_dim == 1 else ins[t][0] + ins[t][1]
            prod = jnp.dot(a, wrefs[t][...], preferred_element_type=jnp.float32)
            acc = prod if acc is None else acc + prod
        acc = act(acc)
        for u in range(nsplit):
            outs[u][...] = acc[:, u * dh:(u + 1) * dh]

    return pl.pallas_call(
        body,
        grid=(N // MM_RB,),
        in_specs=in_specs,
        out_specs=[pl.BlockSpec((MM_RB, dh), lambda i: (i, 0))] * nsplit,
        out_shape=[jax.ShapeDtypeStruct((N, dh), jnp.float32)] * nsplit,
    )(*parts, *ws)


def _add2(p):
    d = p.shape[2]

    def body(pr, o):
        o[...] = pr[0] + pr[1]

    return pl.pallas_call(
        body,
        grid=(N // MM_RB,),
        in_specs=[pl.BlockSpec((2, MM_RB, d), lambda i: (0, i, 0))],
        out_specs=pl.BlockSpec((MM_RB, d), lambda i: (i, 0)),
        out_shape=jax.ShapeDtypeStruct((N, d), jnp.float32),
    )(p)


ADJ_BM, ADJ_BN = 400, 10000


def _adj(zi, zh):
    dims = (((1,), (1,)), ((), ()))

    def body(zi_i, zh_i, zi_j, zh_j, o):
        s1 = lax.dot_general(zi_i[...], zi_j[...], dims,
                             preferred_element_type=jnp.float32)
        s2 = lax.dot_general(zh_i[...], zh_j[...], dims,
                             preferred_element_type=jnp.float32)
        o[...] = jax.nn.sigmoid(s1) + jax.nn.sigmoid(s2)

    return pl.pallas_call(
        body,
        grid=(N // ADJ_BM, N // ADJ_BN),
        in_specs=[
            pl.BlockSpec((ADJ_BM, zi.shape[1]), lambda i, j: (i, 0)),
            pl.BlockSpec((ADJ_BM, zh.shape[1]), lambda i, j: (i, 0)),
            pl.BlockSpec((ADJ_BN, zi.shape[1]), lambda i, j: (j, 0)),
            pl.BlockSpec((ADJ_BN, zh.shape[1]), lambda i, j: (j, 0)),
        ],
        out_specs=pl.BlockSpec((ADJ_BM, ADJ_BN), lambda i, j: (i, j)),
        out_shape=jax.ShapeDtypeStruct((N, N), jnp.float32),
    )(zi, zh, zi, zh)


def kernel(x, edge_index, edge_weight, W1, W2, W3, W4, W5, W6):
    rowx = edge_index[0]
    colx = edge_index[1]
    ew = edge_weight
    tanh = jnp.tanh
    ident = lambda a: a
    w3p = jnp.pad(W3, ((0, 0), (0, 108)))
    w4p = jnp.pad(W4, ((0, 108), (0, 0)))

    t1, = _tc_mm([x[None]], [W1], tanh, 1)
    z1p = _spmm_sc(t1, colx, rowx, ew, 128)
    t2l, t2r = _tc_mm([z1p], [W2], tanh, 2)
    z2lp = _spmm_sc(t2l, colx, rowx, ew, 128)
    z2rp = _spmm_sc(t2r, colx, rowx, ew, 128)
    t3, = _tc_mm([z2lp, z2rp], [w3p[:128], w3p[128:]], ident, 1)
    zip_ = _spmm_sc(t3, colx, rowx, ew, 128)
    t4l, t4r = _tc_mm([zip_], [w4p], tanh, 2)
    h1lp = _spmm_sc(t4l, colx, rowx, ew, 128)
    h1rp = _spmm_sc(t4r, colx, rowx, ew, 128)
    t5, = _tc_mm([h1lp, h1rp], [W5[:128], W5[128:]], tanh, 1)
    h2p = _spmm_sc(t5, colx, rowx, ew, 128)
    t6, = _tc_mm([h2p], [W6], tanh, 1)
    zhp = _spmm_sc(t6, colx, rowx, ew, 128)

    zi_pad = _add2(zip_)
    z_hat = _add2(zhp)
    z_igae = zi_pad[:, :20]
    adj_hat = _adj(zi_pad[:, :32], z_hat)
    return (z_igae, z_hat, adj_hat)
